# baseline (speedup 1.0000x reference)
import functools

import jax
import jax.numpy as jnp
from jax import lax
from jax.experimental import pallas as pl
from jax.experimental.pallas import tpu as pltpu
from jax.experimental.pallas import tpu_sc as plsc

SEQ = 8192
B = 4
D = 1024
NC = 2
NS = 16
NW = NC * NS
ROWS_PW = SEQ // NW
R = 4
NCH = ROWS_PW // R
NB = 2


def _sc_body(x_hbm, pe_hbm, o_hbm, xb, peb, ob, rx, rp, ws):
    wid = lax.axis_index("s") * NC + lax.axis_index("c")
    base = wid * ROWS_PW

    def x_copy(i, slot):
        return pltpu.make_async_copy(
            x_hbm.at[pl.ds(base + i * R, R)], xb.at[slot], rx.at[slot])

    def pe_copy(i, slot):
        return pltpu.make_async_copy(
            pe_hbm.at[pl.ds(base + i * R, R)], peb.at[slot], rp.at[slot])

    def o_copy(i, slot):
        return pltpu.make_async_copy(
            ob.at[slot], o_hbm.at[pl.ds(base + i * R, R)], ws.at[slot])

    for i in range(NB - 1):
        x_copy(i, i).start()
        pe_copy(i, i).start()

    def step(i, carry):
        slot = lax.rem(i, NB)
        nxt = i + NB - 1
        nslot = lax.rem(nxt, NB)

        @pl.when(nxt < NCH)
        def _():
            x_copy(nxt, nslot).start()
            pe_copy(nxt, nslot).start()

        x_copy(i, slot).wait()
        pe_copy(i, slot).wait()

        @pl.when(i >= NB)
        def _():
            o_copy(i - NB, slot).wait()

        def row(r, carry2):
            def col(j, carry3):
                pe_v = peb[slot, r, pl.ds(j * 16, 16)]
                for b in range(B):
                    ob[slot, r, b, pl.ds(j * 16, 16)] = (
                        xb[slot, r, b, pl.ds(j * 16, 16)] + pe_v)
                return carry3
            return lax.fori_loop(0, D // 16, col, carry2)

        lax.fori_loop(0, R, row, 0)
        o_copy(i, slot).start()
        return carry

    lax.fori_loop(0, NCH, step, 0)

    for k in range(NB):
        i = NCH - NB + k
        o_copy(i, i % NB).wait()


def kernel(x, pos_emb):
    seq_len, batch, d_model = x.shape
    sc = pl.kernel(
        _sc_body,
        out_type=jax.ShapeDtypeStruct((seq_len, batch, d_model), x.dtype),
        mesh=plsc.VectorSubcoreMesh(core_axis_name="c", subcore_axis_name="s"),
        scratch_types=[
            pltpu.VMEM((NB, R, B, D), x.dtype),
            pltpu.VMEM((NB, R, D), x.dtype),
            pltpu.VMEM((NB, R, B, D), x.dtype),
            pltpu.SemaphoreType.DMA((NB,)),
            pltpu.SemaphoreType.DMA((NB,)),
            pltpu.SemaphoreType.DMA((NB,)),
        ],
    )
    return sc(x, pos_emb)
